# baseline (speedup 1.0000x reference)
import functools

import jax
import jax.numpy as jnp
from jax import lax
from jax.experimental import pallas as pl
from jax.experimental.pallas import tpu as pltpu
from jax.experimental.pallas import tpu_sc as plsc

N = 10000
E = 160000
IN = 128
HID = 128
K = 10
SEQ = 4

NTILES = 32
ROWS = 320
NPAD = NTILES * ROWS
CH = 128
NCH = E // CH
NOFF = 48
NBUF = 3


def _make_diffusion(W, CHN, NCHN):
    NSUB = W // HID

    def body(feat_hbm, edata_hbm, wdat_hbm, offs_hbm, out_hbm,
             acc, rows0, rows1, rows2, ib0, ib1, ib2,
             wb0, wb1, wb2, offv, g0, g1, g2, s0, s1, s2,
             i0s, i1s, i2s):
        cid = lax.axis_index("c")
        sid = lax.axis_index("s")
        wid = cid * 16 + sid
        rows = (rows0, rows1, rows2)
        ibs = (ib0, ib1, ib2)
        wbs = (wb0, wb1, wb2)
        gsem = (g0, g1, g2)
        ssem = (s0, s1, s2)
        isem = (i0s, i1s, i2s)

        zero16 = jnp.zeros((16,), jnp.float32)

        def _zero_row(i, _):
            for j in range(W // 16):
                if NSUB == 1:
                    rows0[i, pl.ds(j * 16, 16)] = zero16
                else:
                    rows0[i, j // (HID // 16),
                          pl.ds((j % (HID // 16)) * 16, 16)] = zero16
            return 0

        lax.fori_loop(0, CHN, _zero_row, 0)
        abase = sid * ROWS
        pos = 0
        while pos + CHN <= ROWS:
            pltpu.sync_copy(rows0, acc.at[pl.ds(abase + pos, CHN)])
            pos += CHN
        if pos < ROWS:
            pltpu.sync_copy(rows0.at[pl.ds(0, ROWS - pos)],
                            acc.at[pl.ds(abase + pos, ROWS - pos)])

        pltpu.sync_copy(offs_hbm, offv)
        ov = offv[pl.ds(wid, 16)]
        start = ov[0]
        end = ov[1]

        c0 = start // CHN
        c1 = (end + CHN - 1) // CHN
        n = c1 - c0

        def fetch_idx(b, i):
            pltpu.async_copy(edata_hbm.at[c0 + i], ibs[b], isem[b])
            pltpu.async_copy(wdat_hbm.at[c0 + i], wbs[b], isem[b])

        def start_gather(b):
            pltpu.make_async_copy(edata_hbm.at[c0], ibs[b], isem[b]).wait()
            pltpu.make_async_copy(wdat_hbm.at[c0], wbs[b], isem[b]).wait()
            pltpu.async_copy(feat_hbm.at[ibs[b].at[0]], rows[b], gsem[b])

        def consume(b, i):
            pltpu.make_async_copy(feat_hbm.at[ibs[b].at[0]], rows[b],
                                  gsem[b]).wait()
            bn = (b + 1) % NBUF

            @pl.when(i + 1 < n)
            def _():
                start_gather(bn)

            base = (c0 + i) * CHN

            def _scale_group(g, _):
                gb = g * 16
                wvec = wbs[b][pl.ds(gb, 16)]
                lane = base + gb + lax.iota(jnp.int32, 16)
                wvec = jnp.where((lane >= start) & (lane < end), wvec, 0.0)
                for e in range(16):
                    wb = wvec[jnp.broadcast_to(jnp.int32(e), (16,))]
                    r = gb + e
                    for j in range(W // 16):
                        if NSUB == 1:
                            rows[b][r, pl.ds(j * 16, 16)] = (
                                rows[b][r, pl.ds(j * 16, 16)] * wb)
                        else:
                            u, jj = j // (HID // 16), j % (HID // 16)
                            rows[b][r, u, pl.ds(jj * 16, 16)] = (
                                rows[b][r, u, pl.ds(jj * 16, 16)] * wb)
                return 0

            lax.fori_loop(0, CHN // 16, _scale_group, 0)
            pltpu.async_copy(rows[b], acc.at[ibs[b].at[1]], ssem[b],
                             add=True)

        def wait_scatter(b):
            pltpu.make_async_copy(rows[b], acc.at[ibs[b].at[1]],
                                  ssem[b]).wait()

        @pl.when(0 < n)
        def _():
            fetch_idx(0, 0)

            @pl.when(1 < n)
            def _():
                fetch_idx(1, 1)
            start_gather(0)

        def loop_body(jj, _):
            i0 = jj * NBUF
            for b in range(NBUF):
                i = i0 + b
                br = (b + 2) % NBUF

                @pl.when(i < n)
                def _(b=b, i=i, br=br):
                    consume(b, i)
                    k = i + 2

                    @pl.when(k < n)
                    def _():
                        @pl.when(k >= NBUF)
                        def _():
                            wait_scatter(br)
                        fetch_idx(br, k)
            return 0

        lax.fori_loop(0, (n + NBUF - 1) // NBUF, loop_body, 0)

        for b in range(NBUF):
            @pl.when(b < n)
            def _(b=b):
                wait_scatter(b)

        plsc.subcore_barrier()
        pltpu.sync_copy(acc.at[pl.ds(sid * ROWS, ROWS)],
                        out_hbm.at[pl.ds(wid * ROWS, ROWS)])

    if NSUB == 1:
        rshape = lambda r: (r, HID)
    else:
        rshape = lambda r: (r, NSUB, HID)

    @jax.jit
    def step(feat, edata, wdat, offs):
        mesh = plsc.VectorSubcoreMesh(
            core_axis_name="c", subcore_axis_name="s",
            num_cores=2, num_subcores=16)
        return pl.kernel(
            body,
            out_type=jax.ShapeDtypeStruct(rshape(NPAD), jnp.float32),
            mesh=mesh,
            scratch_types=[
                pltpu.VMEM_SHARED(rshape(16 * ROWS), jnp.float32),
                pltpu.VMEM(rshape(CHN), jnp.float32),
                pltpu.VMEM(rshape(CHN), jnp.float32),
                pltpu.VMEM(rshape(CHN), jnp.float32),
                pltpu.VMEM((2, CHN), jnp.int32),
                pltpu.VMEM((2, CHN), jnp.int32),
                pltpu.VMEM((2, CHN), jnp.int32),
                pltpu.VMEM((CHN,), jnp.float32),
                pltpu.VMEM((CHN,), jnp.float32),
                pltpu.VMEM((CHN,), jnp.float32),
                pltpu.VMEM((NOFF,), jnp.int32),
            ] + [pltpu.SemaphoreType.DMA] * 9,
        )(feat, edata, wdat, offs)

    return step


CH2 = 48
NCH2 = -(-E // CH2)
EPAD2 = NCH2 * CH2

_diffusion_step = _make_diffusion(HID, CH, NCH)
_diffusion_step2 = _make_diffusion(2 * HID, CH2, NCH2)


RBLK = 1280
GRID = NPAD // RBLK


def _px2_body(wx_ref, b_ref, xp_ref, xch_ref, out_ref):
    for half in range(2):
        lo = half * HID
        acc = jnp.broadcast_to(b_ref[0, :], (RBLK, HID))
        acc = acc + jnp.dot(xp_ref[:, lo:lo + HID], wx_ref[0],
                            preferred_element_type=jnp.float32)
        for k in range(K):
            acc = acc + jnp.dot(xch_ref[k, :, lo:lo + HID], wx_ref[k + 1],
                                preferred_element_type=jnp.float32)
        out_ref[:, lo:lo + HID] = acc


def _px2_call(wx, b2, xp, xch):
    blk2 = pl.BlockSpec((RBLK, 2 * HID), lambda i: (i, 0))
    chblk2 = pl.BlockSpec((K, RBLK, 2 * HID), lambda i: (0, i, 0))
    return pl.pallas_call(
        _px2_body,
        grid=(GRID,),
        in_specs=[pl.BlockSpec((K + 1, HID, HID), lambda i: (0, 0, 0)),
                  pl.BlockSpec((1, HID), lambda i: (0, 0)), blk2, chblk2],
        out_specs=blk2,
        out_shape=jax.ShapeDtypeStruct((NPAD, 2 * HID), jnp.float32),
    )(wx, b2, xp, xch)


def _gate_body(wh_ref, px_ref, h_ref, hch_ref, ru_ref, rh_ref):
    acc = px_ref[...]
    acc = acc + jnp.dot(h_ref[...], wh_ref[0],
                        preferred_element_type=jnp.float32)
    for k in range(K):
        acc = acc + jnp.dot(hch_ref[k], wh_ref[k + 1],
                            preferred_element_type=jnp.float32)
    ru = jax.nn.sigmoid(acc)
    ru_ref[...] = ru
    rh_ref[...] = ru * h_ref[...]


def _gate_call(wh, px, h, hch):
    blk = pl.BlockSpec((RBLK, HID), lambda i: (i, 0))
    chblk = pl.BlockSpec((K, RBLK, HID), lambda i: (0, i, 0))
    return pl.pallas_call(
        _gate_body,
        grid=(GRID,),
        in_specs=[pl.BlockSpec((K + 1, HID, HID), lambda i: (0, 0, 0)),
                  blk, blk, chblk],
        out_specs=[blk, blk],
        out_shape=[jax.ShapeDtypeStruct((NPAD, HID), jnp.float32),
                   jax.ShapeDtypeStruct((NPAD, HID), jnp.float32)],
    )(wh, px, h, hch)


def _cand_body(wh_ref, px_ref, h_ref, ru_ref, rh_ref, rhch_ref, out_ref):
    acc = px_ref[...]
    acc = acc + jnp.dot(rh_ref[...], wh_ref[0],
                        preferred_element_type=jnp.float32)
    for k in range(K):
        acc = acc + jnp.dot(rhch_ref[k], wh_ref[k + 1],
                            preferred_element_type=jnp.float32)
    c = jax.nn.sigmoid(acc)
    ru = ru_ref[...]
    out_ref[...] = ru * h_ref[...] + (1.0 - ru) * c


def _cand_call(wh, px, h, ru, rh, rhch):
    blk = pl.BlockSpec((RBLK, HID), lambda i: (i, 0))
    chblk = pl.BlockSpec((K, RBLK, HID), lambda i: (0, i, 0))
    return pl.pallas_call(
        _cand_body,
        grid=(GRID,),
        in_specs=[pl.BlockSpec((K + 1, HID, HID), lambda i: (0, 0, 0)),
                  blk, blk, blk, blk, chblk],
        out_specs=blk,
        out_shape=jax.ShapeDtypeStruct((NPAD, HID), jnp.float32),
    )(wh, px, h, ru, rh, rhch)


def kernel(input, hidden, edge_index, edge_weight, W, b):
    src, dst = edge_index[0], edge_index[1]

    order = jnp.argsort(dst)
    dsts = dst[order]
    srcs = src[order]
    wsorted = edge_weight[order]
    wdat = wsorted.reshape(NCH, CH)
    dstl = (dsts % (16 * ROWS)).astype(jnp.int32)
    edata = jnp.stack([srcs.reshape(NCH, CH), dstl.reshape(NCH, CH)], axis=1)
    bounds = (jnp.arange(NOFF, dtype=jnp.int32) * ROWS).clip(max=NPAD)
    offs = jnp.searchsorted(dsts, bounds).astype(jnp.int32)

    epad = ((0, EPAD2 - E),)
    srcs2 = jnp.pad(srcs, epad).reshape(NCH2, CH2)
    dstl2 = jnp.pad(dstl, epad).reshape(NCH2, CH2)
    wdat2 = jnp.pad(wsorted, epad).reshape(NCH2, CH2)
    edata2 = jnp.stack([srcs2, dstl2], axis=1)

    w3 = W.reshape(K + 1, IN + HID, HID)
    wx = w3[:, :IN, :]
    wh = w3[:, IN:, :]
    b2 = b.reshape(1, HID)

    pad_n = ((0, NPAD - N), (0, 0))
    xs4 = jnp.pad(input, ((0, 0),) + pad_n)
    h0 = jnp.pad(hidden[0], pad_n)

    def chain(feat0):
        def body(f, _):
            fn = _diffusion_step(f, edata, wdat, offs)
            return fn, fn
        _, ys = lax.scan(body, feat0, None, length=K)
        return ys

    def chain2(feat0):
        def body(f, _):
            fn = _diffusion_step2(f, edata2, wdat2, offs)
            return fn, fn
        _, ys = lax.scan(body, feat0.reshape(NPAD, 2, HID), None, length=K)
        return ys.reshape(K, NPAD, 2 * HID)

    xpairs = jnp.concatenate([xs4[0::2], xs4[1::2]], axis=2)

    def px_step(_, xp):
        xch = chain2(xp)
        return 0, _px2_call(wx, b2, xp, xch)

    _, pxp = lax.scan(px_step, 0, xpairs)
    pxs = jnp.stack([pxp[0, :, :HID], pxp[0, :, HID:],
                     pxp[1, :, :HID], pxp[1, :, HID:]])

    def tstep(h, px_t):
        hch = chain(h)
        ru, rh = _gate_call(wh, px_t, h, hch)
        rhch = chain(rh)
        hn = _cand_call(wh, px_t, h, ru, rh, rhch)
        return hn, hn

    h_fin, outs = lax.scan(tstep, h0, pxs)

    output = outs[:, :N, :]
    hidden_out = h_fin[:N][None, :, :]
    return (output, hidden_out)


# speedup vs baseline: 1.0356x; 1.0356x over previous
import functools

import jax
import jax.numpy as jnp
from jax import lax
from jax.experimental import pallas as pl
from jax.experimental.pallas import tpu as pltpu
from jax.experimental.pallas import tpu_sc as plsc

N = 10000
E = 160000
IN = 128
HID = 128
K = 10
SEQ = 4

NTILES = 32
ROWS = 320
NPAD = NTILES * ROWS
CH = 128
NCH = E // CH
NOFF = 48
NBUF = 3


def _make_diffusion(W, CHN, NCHN):
    NSUB = W // HID

    def body(feat_hbm, edata_hbm, wdat_hbm, offs_hbm, out_hbm,
             acc, rows0, rows1, rows2, ib0, ib1, ib2,
             wb0, wb1, wb2, offv, g0, g1, g2, s0, s1, s2,
             i0s, i1s, i2s):
        cid = lax.axis_index("c")
        sid = lax.axis_index("s")
        wid = cid * 16 + sid
        rows = (rows0, rows1, rows2)
        ibs = (ib0, ib1, ib2)
        wbs = (wb0, wb1, wb2)
        gsem = (g0, g1, g2)
        ssem = (s0, s1, s2)
        isem = (i0s, i1s, i2s)

        zero16 = jnp.zeros((16,), jnp.float32)

        def _zero_row(i, _):
            for j in range(W // 16):
                if NSUB == 1:
                    rows0[i, pl.ds(j * 16, 16)] = zero16
                else:
                    rows0[i, j // (HID // 16),
                          pl.ds((j % (HID // 16)) * 16, 16)] = zero16
            return 0

        lax.fori_loop(0, CHN, _zero_row, 0)
        abase = sid * ROWS
        pos = 0
        while pos + CHN <= ROWS:
            pltpu.sync_copy(rows0, acc.at[pl.ds(abase + pos, CHN)])
            pos += CHN
        if pos < ROWS:
            pltpu.sync_copy(rows0.at[pl.ds(0, ROWS - pos)],
                            acc.at[pl.ds(abase + pos, ROWS - pos)])

        pltpu.sync_copy(offs_hbm, offv)
        ov = offv[pl.ds(wid, 16)]
        start = ov[0]
        end = ov[1]

        c0 = start // CHN
        c1 = (end + CHN - 1) // CHN
        n = c1 - c0

        def fetch_idx(b, i):
            pltpu.async_copy(edata_hbm.at[c0 + i], ibs[b], isem[b])
            pltpu.async_copy(wdat_hbm.at[c0 + i], wbs[b], isem[b])

        def start_gather(b):
            pltpu.make_async_copy(edata_hbm.at[c0], ibs[b], isem[b]).wait()
            pltpu.make_async_copy(wdat_hbm.at[c0], wbs[b], isem[b]).wait()
            pltpu.async_copy(feat_hbm.at[ibs[b].at[0]], rows[b], gsem[b])

        def consume(b, i):
            pltpu.make_async_copy(feat_hbm.at[ibs[b].at[0]], rows[b],
                                  gsem[b]).wait()
            bn = (b + 1) % NBUF

            @pl.when(i + 1 < n)
            def _():
                start_gather(bn)

            base = (c0 + i) * CHN

            def _scale_group(g, _):
                gb = g * 16
                wvec = wbs[b][pl.ds(gb, 16)]
                lane = base + gb + lax.iota(jnp.int32, 16)
                wvec = jnp.where((lane >= start) & (lane < end), wvec, 0.0)
                for e in range(16):
                    wb = wvec[jnp.broadcast_to(jnp.int32(e), (16,))]
                    r = gb + e
                    for j in range(W // 16):
                        if NSUB == 1:
                            rows[b][r, pl.ds(j * 16, 16)] = (
                                rows[b][r, pl.ds(j * 16, 16)] * wb)
                        else:
                            u, jj = j // (HID // 16), j % (HID // 16)
                            rows[b][r, u, pl.ds(jj * 16, 16)] = (
                                rows[b][r, u, pl.ds(jj * 16, 16)] * wb)
                return 0

            lax.fori_loop(0, CHN // 16, _scale_group, 0)
            pltpu.async_copy(rows[b], acc.at[ibs[b].at[1]], ssem[b],
                             add=True)

        def wait_scatter(b):
            pltpu.make_async_copy(rows[b], acc.at[ibs[b].at[1]],
                                  ssem[b]).wait()

        @pl.when(0 < n)
        def _():
            fetch_idx(0, 0)

            @pl.when(1 < n)
            def _():
                fetch_idx(1, 1)
            start_gather(0)

        def loop_body(jj, _):
            i0 = jj * NBUF
            for b in range(NBUF):
                i = i0 + b
                br = (b + 2) % NBUF

                @pl.when(i < n)
                def _(b=b, i=i, br=br):
                    consume(b, i)
                    k = i + 2

                    @pl.when(k < n)
                    def _():
                        @pl.when(k >= NBUF)
                        def _():
                            wait_scatter(br)
                        fetch_idx(br, k)
            return 0

        lax.fori_loop(0, (n + NBUF - 1) // NBUF, loop_body, 0)

        for b in range(NBUF):
            @pl.when(b < n)
            def _(b=b):
                wait_scatter(b)

        plsc.subcore_barrier()
        pltpu.sync_copy(acc.at[pl.ds(sid * ROWS, ROWS)],
                        out_hbm.at[pl.ds(wid * ROWS, ROWS)])

    if NSUB == 1:
        rshape = lambda r: (r, HID)
    else:
        rshape = lambda r: (r, NSUB, HID)

    @jax.jit
    def step(feat, edata, wdat, offs):
        mesh = plsc.VectorSubcoreMesh(
            core_axis_name="c", subcore_axis_name="s",
            num_cores=2, num_subcores=16)
        return pl.kernel(
            body,
            out_type=jax.ShapeDtypeStruct(rshape(NPAD), jnp.float32),
            mesh=mesh,
            scratch_types=[
                pltpu.VMEM_SHARED(rshape(16 * ROWS), jnp.float32),
                pltpu.VMEM(rshape(CHN), jnp.float32),
                pltpu.VMEM(rshape(CHN), jnp.float32),
                pltpu.VMEM(rshape(CHN), jnp.float32),
                pltpu.VMEM((2, CHN), jnp.int32),
                pltpu.VMEM((2, CHN), jnp.int32),
                pltpu.VMEM((2, CHN), jnp.int32),
                pltpu.VMEM((CHN,), jnp.float32),
                pltpu.VMEM((CHN,), jnp.float32),
                pltpu.VMEM((CHN,), jnp.float32),
                pltpu.VMEM((NOFF,), jnp.int32),
            ] + [pltpu.SemaphoreType.DMA] * 9,
        )(feat, edata, wdat, offs)

    return step


_diffusion_step = _make_diffusion(HID, CH, NCH)


RBLK = 1280
GRID = NPAD // RBLK


def _px_body(wx_ref, b_ref, x0_ref, xch_ref, out_ref):
    acc = jnp.broadcast_to(b_ref[0, :], (RBLK, HID))
    acc = acc + jnp.dot(x0_ref[...], wx_ref[0],
                        preferred_element_type=jnp.float32)
    for k in range(K):
        acc = acc + jnp.dot(xch_ref[k], wx_ref[k + 1],
                            preferred_element_type=jnp.float32)
    out_ref[...] = acc


def _px_call(wx, b2, x0, xch):
    blk = pl.BlockSpec((RBLK, HID), lambda i: (i, 0))
    chblk = pl.BlockSpec((K, RBLK, HID), lambda i: (0, i, 0))
    return pl.pallas_call(
        _px_body,
        grid=(GRID,),
        in_specs=[pl.BlockSpec((K + 1, HID, HID), lambda i: (0, 0, 0)),
                  pl.BlockSpec((1, HID), lambda i: (0, 0)), blk, chblk],
        out_specs=blk,
        out_shape=jax.ShapeDtypeStruct((NPAD, HID), jnp.float32),
    )(wx, b2, x0, xch)


def _gate_body(wh_ref, px_ref, h_ref, hch_ref, ru_ref, rh_ref):
    acc = px_ref[...]
    acc = acc + jnp.dot(h_ref[...], wh_ref[0],
                        preferred_element_type=jnp.float32)
    for k in range(K):
        acc = acc + jnp.dot(hch_ref[k], wh_ref[k + 1],
                            preferred_element_type=jnp.float32)
    ru = jax.nn.sigmoid(acc)
    ru_ref[...] = ru
    rh_ref[...] = ru * h_ref[...]


def _gate_call(wh, px, h, hch):
    blk = pl.BlockSpec((RBLK, HID), lambda i: (i, 0))
    chblk = pl.BlockSpec((K, RBLK, HID), lambda i: (0, i, 0))
    return pl.pallas_call(
        _gate_body,
        grid=(GRID,),
        in_specs=[pl.BlockSpec((K + 1, HID, HID), lambda i: (0, 0, 0)),
                  blk, blk, chblk],
        out_specs=[blk, blk],
        out_shape=[jax.ShapeDtypeStruct((NPAD, HID), jnp.float32),
                   jax.ShapeDtypeStruct((NPAD, HID), jnp.float32)],
    )(wh, px, h, hch)


def _cand_body(wh_ref, px_ref, h_ref, ru_ref, rh_ref, rhch_ref, out_ref):
    acc = px_ref[...]
    acc = acc + jnp.dot(rh_ref[...], wh_ref[0],
                        preferred_element_type=jnp.float32)
    for k in range(K):
        acc = acc + jnp.dot(rhch_ref[k], wh_ref[k + 1],
                            preferred_element_type=jnp.float32)
    c = jax.nn.sigmoid(acc)
    ru = ru_ref[...]
    out_ref[...] = ru * h_ref[...] + (1.0 - ru) * c


def _cand_call(wh, px, h, ru, rh, rhch):
    blk = pl.BlockSpec((RBLK, HID), lambda i: (i, 0))
    chblk = pl.BlockSpec((K, RBLK, HID), lambda i: (0, i, 0))
    return pl.pallas_call(
        _cand_body,
        grid=(GRID,),
        in_specs=[pl.BlockSpec((K + 1, HID, HID), lambda i: (0, 0, 0)),
                  blk, blk, blk, blk, chblk],
        out_specs=blk,
        out_shape=jax.ShapeDtypeStruct((NPAD, HID), jnp.float32),
    )(wh, px, h, ru, rh, rhch)


def kernel(input, hidden, edge_index, edge_weight, W, b):
    src, dst = edge_index[0], edge_index[1]

    order = jnp.argsort(dst)
    dsts = dst[order]
    srcs = src[order]
    wsorted = edge_weight[order]
    wdat = wsorted.reshape(NCH, CH)
    dstl = (dsts % (16 * ROWS)).astype(jnp.int32)
    edata = jnp.stack([srcs.reshape(NCH, CH), dstl.reshape(NCH, CH)], axis=1)
    bounds = (jnp.arange(NOFF, dtype=jnp.int32) * ROWS).clip(max=NPAD)
    offs = jnp.searchsorted(dsts, bounds).astype(jnp.int32)

    w3 = W.reshape(K + 1, IN + HID, HID)
    wx = w3[:, :IN, :]
    wh = w3[:, IN:, :]
    b2 = b.reshape(1, HID)

    pad_n = ((0, NPAD - N), (0, 0))
    xs4 = jnp.pad(input, ((0, 0),) + pad_n)
    h0 = jnp.pad(hidden[0], pad_n)

    def chain(feat0):
        def body(f, _):
            fn = _diffusion_step(f, edata, wdat, offs)
            return fn, fn
        _, ys = lax.scan(body, feat0, None, length=K)
        return ys

    def px_step(_, x0):
        xch = chain(x0)
        return 0, _px_call(wx, b2, x0, xch)

    _, pxs = lax.scan(px_step, 0, xs4)

    def tstep(h, px_t):
        hch = chain(h)
        ru, rh = _gate_call(wh, px_t, h, hch)
        rhch = chain(rh)
        hn = _cand_call(wh, px_t, h, ru, rh, rhch)
        return hn, hn

    h_fin, outs = lax.scan(tstep, h0, pxs)

    output = outs[:, :N, :]
    hidden_out = h_fin[:N][None, :, :]
    return (output, hidden_out)
